# baseline (speedup 1.0000x reference)
import jax
import jax.numpy as jnp
from jax import lax
from jax.experimental import pallas as pl
from jax.experimental.pallas import tpu as pltpu

F32 = jnp.float32


def _dot(a, b):
    return jnp.dot(a, b, preferred_element_type=F32)


def _dot_t(a, b):
    return lax.dot_general(a, b, (((1,), (1,)), ((), ())),
                           preferred_element_type=F32)


def _xla_sum(x):
    w = x.shape[-1]
    acc = x[:, 0:128]
    for i in range(1, w // 128):
        acc = acc + x[:, i * 128:(i + 1) * 128]
    s = acc[:, 0:8]
    for i in range(1, 16):
        s = s + acc[:, i * 8:(i + 1) * 8]
    s = s[:, 0:4] + s[:, 4:8]
    s = s[:, 0:2] + s[:, 2:4]
    return s[:, 0:1] + s[:, 1:2]


def _ln(x, g, b):
    n = x.shape[-1]
    m = _xla_sum(x) * (1.0 / n)
    d = x - m
    v = _xla_sum(d * d) * (1.0 / n)
    return d / jnp.sqrt(v + 1e-5) * g + b


def _att_softmax_v(sc, vh):
    mx = jnp.max(sc, axis=-1, keepdims=True)
    e = jnp.exp(sc - mx)
    return _dot(e, vh) / _xla_sum(e)


def _conv_kernel(tap_ref, w_ref, b_ref, y_ref):
    y_ref[0] = _dot(tap_ref[0], w_ref[...]) + b_ref[...]


def _make_enc_kernel(Tp, D, H, hd, K, nlayer):
    def enc_kernel(lens_ref, y_ref, mean_ref, var_ref, g_ref, b_ref,
                   cb_ref, cn_ref, *rest):
        layer_refs = rest[:-2]
        vq_ref, idx_ref = rest[-2:]
        b = pl.program_id(0)
        len_b = lens_ref[b]
        x = (y_ref[0] - mean_ref[...]) / jnp.sqrt(var_ref[...] + 1e-5) \
            * g_ref[...] + b_ref[...]
        x = jnp.maximum(x, 0.0)
        km = lax.broadcasted_iota(jnp.int32, (1, Tp), 1) < len_b
        per = 12
        for l in range(nlayer):
            (wq3, wk3, wv3, wo, ln1g, ln1b, w1, b1, w2, b2, ln2g, ln2b) = \
                layer_refs[l * per:(l + 1) * per]
            outs = []
            for h in range(H):
                qh = _dot(x, wq3[h])
                kh = _dot(x, wk3[h])
                vh = _dot(x, wv3[h])
                sc = _dot_t(qh, kh) / jnp.sqrt(jnp.float32(hd))
                sc = jnp.where(km, sc, -1e9)
                outs.append(_att_softmax_v(sc, vh))
            o = _dot(jnp.concatenate(outs, axis=1), wo[...])
            x = _ln(x + o, ln1g[...], ln1b[...])
            f = jnp.maximum(_dot(x, w1[...]) + b1[...], 0.0)
            f = _dot(f, w2[...]) + b2[...]
            x = _ln(x + f, ln2g[...], ln2b[...])
        rowm = lax.broadcasted_iota(jnp.int32, (Tp, 1), 0) < len_b
        x = jnp.where(rowm, x, 0.0)
        zn = _xla_sum(x * x)
        dist = zn + cn_ref[...] - 2.0 * _dot_t(x, cb_ref[...])
        mn = jnp.min(dist, axis=1, keepdims=True)
        col = lax.broadcasted_iota(jnp.int32, (Tp, K), 1)
        idxk = jnp.min(jnp.where(dist == mn, col, K), axis=1, keepdims=True)
        onehot = (col == idxk).astype(F32)
        quant = jnp.dot(onehot, cb_ref[...], precision=lax.Precision.HIGHEST,
                        preferred_element_type=F32)
        vq_ref[0] = x + (quant - x)
        idx_ref[0] = idxk
    return enc_kernel


def kernel(feature, data_len, params):
    p = params
    B, T, NMEL = feature.shape
    D = p['conv_w'].shape[0]
    K = p['codebook'].shape[0]
    H = 4
    hd = D // H
    Tp = T // 2
    nlayer = len(p['layers'])

    xp = jnp.pad(feature, ((0, 0), (1, 3), (0, 0)))
    xr = xp.reshape(B, (T + 4) // 2, 2, NMEL)
    xe, xo = xr[:, :, 0, :], xr[:, :, 1, :]
    taps = jnp.concatenate([
        xe[:, 0:Tp], xo[:, 0:Tp], xe[:, 1:Tp + 1], xo[:, 1:Tp + 1],
        xe[:, 2:Tp + 2]], axis=-1)
    wt = jnp.transpose(p['conv_w'], (2, 1, 0)).reshape(5 * NMEL, D)
    conv_b = p['conv_b'].reshape(1, D)

    conv_out = pl.pallas_call(
        _conv_kernel,
        grid=(B,),
        in_specs=[
            pl.BlockSpec((1, Tp, 5 * NMEL), lambda b: (b, 0, 0)),
            pl.BlockSpec((5 * NMEL, D), lambda b: (0, 0)),
            pl.BlockSpec((1, D), lambda b: (0, 0)),
        ],
        out_specs=pl.BlockSpec((1, Tp, D), lambda b: (b, 0, 0)),
        out_shape=jax.ShapeDtypeStruct((B, Tp, D), F32),
    )(taps, wt, conv_b)

    mean = jnp.mean(conv_out, axis=(0, 1)).reshape(1, D)
    var = jnp.var(conv_out, axis=(0, 1)).reshape(1, D)
    bn_g = p['bn_g'].reshape(1, D)
    bn_b = p['bn_b'].reshape(1, D)
    lens = ((jnp.maximum(data_len, 1) + 1) // 2).astype(jnp.int32)
    cb = p['codebook']
    cn = jnp.sum(cb ** 2, axis=1)[None, :]

    layer_args = []
    for lp in p['layers']:
        wq3 = lp['wq'].reshape(D, H, hd).transpose(1, 0, 2)
        wk3 = lp['wk'].reshape(D, H, hd).transpose(1, 0, 2)
        wv3 = lp['wv'].reshape(D, H, hd).transpose(1, 0, 2)
        layer_args += [wq3, wk3, wv3, lp['wo'],
                       lp['ln1_g'].reshape(1, D), lp['ln1_b'].reshape(1, D),
                       lp['w1'], lp['b1'].reshape(1, -1),
                       lp['w2'], lp['b2'].reshape(1, D),
                       lp['ln2_g'].reshape(1, D), lp['ln2_b'].reshape(1, D)]

    full = lambda a: pl.BlockSpec(a.shape, lambda b: (0,) * a.ndim)

    enc = _make_enc_kernel(Tp, D, H, hd, K, nlayer)
    vq, idxo = pl.pallas_call(
        enc,
        grid=(B,),
        in_specs=[
            pl.BlockSpec(memory_space=pltpu.SMEM),
            pl.BlockSpec((1, Tp, D), lambda b: (b, 0, 0)),
            full(mean), full(var), full(bn_g), full(bn_b),
            full(cb), full(cn),
            *[full(a) for a in layer_args],
        ],
        out_specs=[
            pl.BlockSpec((1, Tp, D), lambda b: (b, 0, 0)),
            pl.BlockSpec((1, Tp, 1), lambda b: (b, 0, 0)),
        ],
        out_shape=[
            jax.ShapeDtypeStruct((B, Tp, D), F32),
            jax.ShapeDtypeStruct((B, Tp, 1), jnp.int32),
        ],
    )(lens, conv_out, mean, var, bn_g, bn_b, cb, cn, *layer_args)

    return vq, idxo[:, :, 0], cb
